# baseline (speedup 1.0000x reference)
import functools

import numpy as np

import jax
import jax.numpy as jnp
from jax import lax
from jax.experimental import pallas as pl
from jax.experimental.pallas import tpu as pltpu
from jax.experimental.pallas import tpu_sc as plsc

NUM_ENT = 10000
NUM_REL = 200
DIM = 128
E = 320000
B = 1024
K_H, K_W, KER, NF = 8, 16, 7, 200
FLAT_H = 2 * K_H - KER + 1
FLAT_W = K_W - KER + 1
NPOS = FLAT_H * FLAT_W
FLAT = NPOS * NF
KK = KER * KER

NC, NS = 2, 16
E2 = E // 2
EPT = E2 // NS
CH = 48
NCHUNK = 210
EPT_P = NCHUNK * CH
E2_P = NS * EPT_P
ROWS_PT = 624
ROWS_TAIL = NUM_ENT - NS * ROWS_PT

BBLK = 64
NBB = B // BBLK
NBLK = 2048
NNB = (NUM_ENT + NBLK - 1) // NBLK

_EPS = 1e-5


def _patch_matrix():
    p = np.zeros((2 * DIM, NPOS, KK), np.float32)
    for i in range(FLAT_H):
        for j in range(FLAT_W):
            pos = i * FLAT_W + j
            for dp in range(KER):
                for dq in range(KER):
                    c = (i + dp) * K_W + (j + dq)
                    p[c, pos, dp * KER + dq] = 1.0
    return p


_PATCH = _patch_matrix()


def _edge_body(x_hbm, r_hbm, src_hbm, et_hbm, dst_hbm, nrm_hbm, zero_hbm,
               out_hbm, acc, xrows, rrows, msg, sidx, eidx, didx, nvm,
               semi, semg):
    c = lax.axis_index("c")
    s = lax.axis_index("s")
    rb = pl.multiple_of(s * ROWS_PT, 8)
    pltpu.sync_copy(zero_hbm.at[pl.ds(rb, ROWS_PT)], acc.at[pl.ds(rb, ROWS_PT)])

    @pl.when(s == 0)
    def _():
        tb = NS * ROWS_PT
        pltpu.sync_copy(zero_hbm.at[pl.ds(tb, ROWS_TAIL)],
                        acc.at[pl.ds(tb, ROWS_TAIL)])
    plsc.subcore_barrier()

    tb = pl.multiple_of((c * NS + s) * EPT_P, 8)

    dn = lax.GatherDimensionNumbers(
        offset_dims=(), collapsed_slice_dims=(0,), start_index_map=(0,))

    def compute(p):
        def edge16(i0, lanes):
            nv16 = nvm[p, pl.ds(i0, 16)]
            for kk in lanes:
                i = i0 + kk
                bc = lax.gather(
                    nv16, jnp.full((16, 1), kk, jnp.int32), dn, (1,),
                    mode=lax.GatherScatterMode.PROMISE_IN_BOUNDS)
                for j in range(DIM // 16):
                    sl = pl.ds(j * 16, 16)
                    msg[i, sl] = xrows[p, i, sl] * rrows[p, i, sl] * bc

        def grp(g, c2):
            edge16(g * 16, range(16))
            return c2
        lax.fori_loop(0, CH // 16, grp, 0)
        if CH % 16:
            edge16(CH - 16, range(16 - CH % 16, 16))
        pltpu.sync_copy(msg, acc.at[didx.at[p]], add=True)

    def pair(t, carry):
        ci = []
        for p in range(2):
            base = pl.multiple_of(tb + (2 * t + p) * CH, 8)
            ci.append(pltpu.async_copy(src_hbm.at[pl.ds(base, CH)],
                                       sidx.at[p], semi))
            ci.append(pltpu.async_copy(et_hbm.at[pl.ds(base, CH)],
                                       eidx.at[p], semi))
            ci.append(pltpu.async_copy(dst_hbm.at[pl.ds(base, CH)],
                                       didx.at[p], semi))
            ci.append(pltpu.async_copy(nrm_hbm.at[pl.ds(base, CH)],
                                       nvm.at[p], semi))
        for h in ci:
            h.wait()
        ga0 = pltpu.async_copy(x_hbm.at[sidx.at[0]], xrows.at[0], semg)
        gr0 = pltpu.async_copy(r_hbm.at[eidx.at[0]], rrows.at[0], semg)
        ga1 = pltpu.async_copy(x_hbm.at[sidx.at[1]], xrows.at[1], semg)
        gr1 = pltpu.async_copy(r_hbm.at[eidx.at[1]], rrows.at[1], semg)
        ga0.wait()
        gr0.wait()
        compute(0)
        ga1.wait()
        gr1.wait()
        compute(1)
        return carry

    lax.fori_loop(0, NCHUNK // 2, pair, 0)
    plsc.subcore_barrier()
    ob = pl.multiple_of(c * NUM_ENT + s * ROWS_PT, 8)
    pltpu.sync_copy(acc.at[pl.ds(rb, ROWS_PT)], out_hbm.at[pl.ds(ob, ROWS_PT)])

    @pl.when(s == 0)
    def _():
        tb = NS * ROWS_PT
        otb = pl.multiple_of(c * NUM_ENT + tb, 8)
        pltpu.sync_copy(acc.at[pl.ds(tb, ROWS_TAIL)],
                        out_hbm.at[pl.ds(otb, ROWS_TAIL)])


@functools.lru_cache(maxsize=None)
def _edge_pass_fn():
    return pl.kernel(
        _edge_body,
        out_type=jax.ShapeDtypeStruct((2 * NUM_ENT, DIM), jnp.float32),
        mesh=plsc.VectorSubcoreMesh(core_axis_name="c", subcore_axis_name="s",
                                    num_cores=NC, num_subcores=NS),
        scratch_types=[
            pltpu.VMEM_SHARED((NUM_ENT, DIM), jnp.float32),
            pltpu.VMEM((2, CH, DIM), jnp.float32),
            pltpu.VMEM((2, CH, DIM), jnp.float32),
            pltpu.VMEM((CH, DIM), jnp.float32),
            pltpu.VMEM((2, CH), jnp.int32),
            pltpu.VMEM((2, CH), jnp.int32),
            pltpu.VMEM((2, CH), jnp.int32),
            pltpu.VMEM((2, CH), jnp.float32),
            pltpu.SemaphoreType.DMA,
            pltpu.SemaphoreType.DMA,
        ],
    )


def _edge_pass(*args):
    return _edge_pass_fn()(*args)


GPT = B // (NC * NS)


def _gather_body(x_hbm, r_hbm, subj_hbm, rel_hbm, sub_out, rel_out, idxv, rows, sem):
    c = lax.axis_index("c")
    s = lax.axis_index("s")
    base = pl.multiple_of((s * NC + c) * GPT, 8)
    pltpu.sync_copy(subj_hbm.at[pl.ds(base, GPT)], idxv)
    pltpu.async_copy(x_hbm.at[idxv], rows, sem).wait()
    pltpu.sync_copy(rows, sub_out.at[pl.ds(base, GPT)])
    pltpu.sync_copy(rel_hbm.at[pl.ds(base, GPT)], idxv)
    pltpu.async_copy(r_hbm.at[idxv], rows, sem).wait()
    pltpu.sync_copy(rows, rel_out.at[pl.ds(base, GPT)])


@functools.lru_cache(maxsize=None)
def _gather_pass_fn():
    return pl.kernel(
        _gather_body,
        out_type=[jax.ShapeDtypeStruct((B, DIM), jnp.float32),
                  jax.ShapeDtypeStruct((B, DIM), jnp.float32)],
        mesh=plsc.VectorSubcoreMesh(core_axis_name="c", subcore_axis_name="s",
                                    num_cores=NC, num_subcores=NS),
        scratch_types=[
            pltpu.VMEM((GPT,), jnp.int32),
            pltpu.VMEM((GPT, DIM), jnp.float32),
            pltpu.SemaphoreType.DMA,
        ],
    )


def _gather_pass(*args):
    return _gather_pass_fn()(*args)


def _dense_body(ain, aout, x, r, win, wout, wloop, wrel, lrel, b, bns, bno,
                xo, ro):
    f32 = jnp.float32
    agg = jnp.dot(ain[...], win[...], preferred_element_type=f32)
    agg = agg + jnp.dot(aout[...], wout[...], preferred_element_type=f32)
    loop = jnp.dot(x[...] * lrel[...], wloop[...], preferred_element_type=f32)
    h = agg * (1.0 / 3.0) + loop * (1.0 / 3.0) + b[...]
    mean = jnp.mean(h, axis=0, keepdims=True)
    var = jnp.mean((h - mean) * (h - mean), axis=0, keepdims=True)
    h = (h - mean) * lax.rsqrt(var + _EPS) * bns[...] + bno[...]
    xo[...] = jnp.tanh(h)
    ro[...] = jnp.dot(r[...], wrel[...], preferred_element_type=f32)


def _dense_layer(ain, aout, x, r, win, wout, wloop, wrel, lrel, b, bns, bno):
    return pl.pallas_call(
        _dense_body,
        out_shape=[jax.ShapeDtypeStruct((NUM_ENT, DIM), jnp.float32),
                   jax.ShapeDtypeStruct((2 * NUM_REL, DIM), jnp.float32)],
    )(ain, aout, x, r, win, wout, wloop, wrel,
      lrel.reshape(1, DIM), b.reshape(1, DIM),
      bns.reshape(1, DIM), bno.reshape(1, DIM))


def _d1_body(img, wbig, pool, stats, gstats):
    i = pl.program_id(0)
    blk = img[...]
    zraw = jnp.dot(blk, wbig[...], preferred_element_type=jnp.float32)
    s1 = jnp.sum(zraw, axis=0, keepdims=True)
    s2 = jnp.sum(zraw * zraw, axis=0, keepdims=True)
    s1f = jnp.dot(s1, pool[...], preferred_element_type=jnp.float32)
    s2f = jnp.dot(s2, pool[...], preferred_element_type=jnp.float32)
    part = jnp.concatenate([s1f, s2f], axis=0)
    gi1 = jnp.sum(blk)
    gi2 = jnp.sum(blk * blk)
    gpart = jnp.concatenate([jnp.full((1, DIM), gi1, jnp.float32),
                             jnp.full((1, DIM), gi2, jnp.float32)], axis=0)

    @pl.when(i == 0)
    def _():
        stats[...] = part
        gstats[...] = gpart

    @pl.when(i > 0)
    def _():
        stats[...] = stats[...] + part
        gstats[...] = gstats[...] + gpart


def _d1_call(img, wbig, pool):
    return pl.pallas_call(
        _d1_body,
        grid=(NBB,),
        in_specs=[pl.BlockSpec((BBLK, 2 * DIM), lambda i: (i, 0)),
                  pl.BlockSpec((2 * DIM, FLAT), lambda i: (0, 0)),
                  pl.BlockSpec((FLAT, NF), lambda i: (0, 0))],
        out_specs=[pl.BlockSpec((2, NF), lambda i: (0, 0)),
                   pl.BlockSpec((2, DIM), lambda i: (0, 0))],
        out_shape=[jax.ShapeDtypeStruct((2, NF), jnp.float32),
                   jax.ShapeDtypeStruct((2, DIM), jnp.float32)],
    )(img, wbig, pool)


def _d1b_body(pool, stats, gstats, ksum, cb, b0s, b0o, bcs, bco,
              scale_o, mix_o, a0_o):
    f32 = jnp.float32
    n0 = float(B * 2 * DIM)
    m0 = gstats[0, 0] / n0
    v0 = gstats[1, 0] / n0 - m0 * m0
    a0 = b0s[0, 0] * lax.rsqrt(v0 + _EPS)
    c0 = b0o[0, 0] - m0 * a0
    beta = c0 * ksum[...] + cb[...]
    nz = float(B * NPOS)
    mraw = stats[0:1, :] / nz
    vraw = stats[1:2, :] / nz - mraw * mraw
    mc = a0 * mraw + beta
    vc = (a0 * a0) * vraw
    scale_c = bcs[...] * lax.rsqrt(vc + _EPS)
    off_c = bco[...] - mc * scale_c
    mix = off_c + beta * scale_c
    scale_o[...] = lax.dot_general(scale_c, pool[...], (((1,), (1,)), ((), ())),
                                   preferred_element_type=f32)
    mix_o[...] = lax.dot_general(mix, pool[...], (((1,), (1,)), ((), ())),
                                 preferred_element_type=f32)
    a0_o[...] = jnp.full((1, DIM), a0, f32)


def _d1b_call(pool, stats, gstats, ksum, cb, b0s, b0o, bcs, bco):
    return pl.pallas_call(
        _d1b_body,
        out_shape=[jax.ShapeDtypeStruct((1, FLAT), jnp.float32),
                   jax.ShapeDtypeStruct((1, FLAT), jnp.float32),
                   jax.ShapeDtypeStruct((1, DIM), jnp.float32)],
    )(pool, stats, gstats, ksum.reshape(1, NF), cb.reshape(1, NF),
      b0s.reshape(1, 1), b0o.reshape(1, 1), bcs.reshape(1, NF),
      bco.reshape(1, NF))


def _d2_body(img, wbig, scale_f, mix_f, a0v, fw, fb, fo):
    f32 = jnp.float32
    blk = img[...]
    zraw = jnp.dot(blk, wbig[...], preferred_element_type=f32)
    z = (a0v[0, 0] * zraw) * scale_f[...] + mix_f[...]
    a = jnp.maximum(z, 0.0)
    fo[...] = jnp.dot(a, fw[...], preferred_element_type=f32) + fb[...]


def _d2_call(img, wbig, scale_f, mix_f, a0v, fw2, fb):
    return pl.pallas_call(
        _d2_body,
        grid=(NBB,),
        in_specs=[pl.BlockSpec((BBLK, 2 * DIM), lambda i: (i, 0)),
                  pl.BlockSpec((2 * DIM, FLAT), lambda i: (0, 0)),
                  pl.BlockSpec((1, FLAT), lambda i: (0, 0)),
                  pl.BlockSpec((1, FLAT), lambda i: (0, 0)),
                  pl.BlockSpec((1, DIM), lambda i: (0, 0)),
                  pl.BlockSpec((FLAT, DIM), lambda i: (0, 0)),
                  pl.BlockSpec((1, DIM), lambda i: (0, 0))],
        out_specs=pl.BlockSpec((BBLK, DIM), lambda i: (i, 0)),
        out_shape=jax.ShapeDtypeStruct((B, DIM), jnp.float32),
    )(img, wbig, scale_f, mix_f, a0v, fw2, fb.reshape(1, DIM))


def _d3_body(f, bfs, bfo, x2, eb, out):
    h = f[...]
    mean = jnp.mean(h, axis=0, keepdims=True)
    var = jnp.mean((h - mean) * (h - mean), axis=0, keepdims=True)
    h = (h - mean) * lax.rsqrt(var + _EPS) * bfs[...] + bfo[...]
    h = jnp.maximum(h, 0.0)
    sc = lax.dot_general(h, x2[...], (((1,), (1,)), ((), ())),
                         preferred_element_type=jnp.float32)
    out[...] = jax.nn.sigmoid(sc + eb[...])


def _d3_call(f, bfs, bfo, x2, ent_bias):
    return pl.pallas_call(
        _d3_body,
        grid=(NNB,),
        in_specs=[pl.BlockSpec((B, DIM), lambda i: (0, 0)),
                  pl.BlockSpec((1, DIM), lambda i: (0, 0)),
                  pl.BlockSpec((1, DIM), lambda i: (0, 0)),
                  pl.BlockSpec((NBLK, DIM), lambda i: (i, 0)),
                  pl.BlockSpec((1, NBLK), lambda i: (0, i))],
        out_specs=pl.BlockSpec((B, NBLK), lambda i: (0, i)),
        out_shape=jax.ShapeDtypeStruct((B, NUM_ENT), jnp.float32),
    )(f, bfs.reshape(1, DIM), bfo.reshape(1, DIM), x2, ent_bias.reshape(1, NUM_ENT))


def kernel(edge_index, edge_type, edge_norm, subj, rel, init_embed, init_rel,
           w_in1, w_out1, w_loop1, w_rel1, loop_rel1, b1, bn1_s, bn1_o,
           w_in2, w_out2, w_loop2, w_rel2, loop_rel2, b2, bn2_s, bn2_o,
           conv_w, conv_b, fc_w, fc_b, bn0_s, bn0_o, bnc_s, bnc_o,
           bnf_s, bnf_o, ent_bias):
    i32 = jnp.int32

    def prep(a, dt):
        ap = jnp.pad(a.astype(dt).reshape(2, E2), ((0, 0), (0, E2_P - E2)))
        return ap.reshape(2 * E2_P)

    src = prep(edge_index[0], i32)
    dst = prep(edge_index[1], i32)
    et = prep(edge_type, i32)
    nrm = prep(edge_norm, jnp.float32)
    zeros = jnp.zeros((NUM_ENT, DIM), jnp.float32)

    a1 = _edge_pass(init_embed, init_rel, src, et, dst, nrm, zeros)
    x1, r1 = _dense_layer(a1[:NUM_ENT], a1[NUM_ENT:], init_embed, init_rel,
                          w_in1, w_out1, w_loop1, w_rel1, loop_rel1, b1,
                          bn1_s, bn1_o)
    a2 = _edge_pass(x1, r1, src, et, dst, nrm, zeros)
    x2, r2 = _dense_layer(a2[:NUM_ENT], a2[NUM_ENT:], x1, r1,
                          w_in2, w_out2, w_loop2, w_rel2, loop_rel2, b2,
                          bn2_s, bn2_o)

    sub_e, rel_e = _gather_pass(x2, r2, subj.astype(i32), rel.astype(i32))
    img = jnp.concatenate([sub_e, rel_e], axis=1)
    km = conv_w.reshape(NF, KK)
    wbig = jnp.einsum("cpk,fk->cpf", _PATCH, km).reshape(2 * DIM, FLAT)
    pool = jnp.tile(jnp.eye(NF, dtype=jnp.float32), (NPOS, 1))
    ksum = jnp.sum(km, axis=1)
    fw2 = fc_w.reshape(NF, NPOS, DIM).transpose(1, 0, 2).reshape(FLAT, DIM)

    stats, gstats = _d1_call(img, wbig, pool)
    scale_f, mix_f, a0v = _d1b_call(pool, stats, gstats, ksum, conv_b,
                                    bn0_s, bn0_o, bnc_s, bnc_o)
    f = _d2_call(img, wbig, scale_f, mix_f, a0v, fw2, fc_b)
    return _d3_call(f, bnf_s, bnf_o, x2, ent_bias)


# speedup vs baseline: 1.3871x; 1.3871x over previous
import functools

import numpy as np

import jax
import jax.numpy as jnp
from jax import lax
from jax.experimental import pallas as pl
from jax.experimental.pallas import tpu as pltpu
from jax.experimental.pallas import tpu_sc as plsc

NUM_ENT = 10000
NUM_REL = 200
DIM = 128
E = 320000
B = 1024
K_H, K_W, KER, NF = 8, 16, 7, 200
FLAT_H = 2 * K_H - KER + 1
FLAT_W = K_W - KER + 1
NPOS = FLAT_H * FLAT_W
FLAT = NPOS * NF
KK = KER * KER

NC, NS = 2, 16
E2 = E // 2
EPT = E2 // NS
CH = 40
NCHUNK = 252
EPT_P = NCHUNK * CH
E2_P = NS * EPT_P
ROWS_PT = 624
ROWS_TAIL = NUM_ENT - NS * ROWS_PT

BBLK = 64
NBB = B // BBLK
NBLK = 2048
NNB = (NUM_ENT + NBLK - 1) // NBLK

_EPS = 1e-5


def _patch_matrix():
    p = np.zeros((2 * DIM, NPOS, KK), np.float32)
    for i in range(FLAT_H):
        for j in range(FLAT_W):
            pos = i * FLAT_W + j
            for dp in range(KER):
                for dq in range(KER):
                    c = (i + dp) * K_W + (j + dq)
                    p[c, pos, dp * KER + dq] = 1.0
    return p


_PATCH = _patch_matrix()


def _edge_body(x_hbm, r_hbm, src_hbm, et_hbm, dst_hbm, nrm_hbm, zero_hbm,
               out_hbm, acc, xrows, rrows, msg, sidx, eidx, didx, nvm,
               semi, semg):
    c = lax.axis_index("c")
    s = lax.axis_index("s")
    rb = pl.multiple_of(s * ROWS_PT, 8)
    pltpu.sync_copy(zero_hbm.at[pl.ds(rb, ROWS_PT)], acc.at[pl.ds(rb, ROWS_PT)])

    @pl.when(s == 0)
    def _():
        tb = NS * ROWS_PT
        pltpu.sync_copy(zero_hbm.at[pl.ds(tb, ROWS_TAIL)],
                        acc.at[pl.ds(tb, ROWS_TAIL)])
    plsc.subcore_barrier()

    tb = pl.multiple_of((c * NS + s) * EPT_P, 8)

    dn = lax.GatherDimensionNumbers(
        offset_dims=(), collapsed_slice_dims=(0,), start_index_map=(0,))

    def compute(p):
        def edge16(i0, lanes):
            nv16 = nvm[p, pl.ds(i0, 16)]
            for kk in lanes:
                i = i0 + kk
                bc = lax.gather(
                    nv16, jnp.full((16, 1), kk, jnp.int32), dn, (1,),
                    mode=lax.GatherScatterMode.PROMISE_IN_BOUNDS)
                for j in range(DIM // 16):
                    sl = pl.ds(j * 16, 16)
                    msg[i, sl] = xrows[p, i, sl] * rrows[p, i, sl] * bc

        def grp(g, c2):
            edge16(g * 16, range(16))
            return c2
        lax.fori_loop(0, CH // 16, grp, 0)
        if CH % 16:
            edge16(CH - 16, range(16 - CH % 16, 16))
        pltpu.sync_copy(msg, acc.at[didx.at[p]], add=True)

    def pair(t, carry):
        ci = []
        for p in range(2):
            base = pl.multiple_of(tb + (2 * t + p) * CH, 8)
            ci.append(pltpu.async_copy(src_hbm.at[pl.ds(base, CH)],
                                       sidx.at[p], semi))
            ci.append(pltpu.async_copy(et_hbm.at[pl.ds(base, CH)],
                                       eidx.at[p], semi))
            ci.append(pltpu.async_copy(dst_hbm.at[pl.ds(base, CH)],
                                       didx.at[p], semi))
            ci.append(pltpu.async_copy(nrm_hbm.at[pl.ds(base, CH)],
                                       nvm.at[p], semi))
        for h in ci:
            h.wait()
        ga0 = pltpu.async_copy(x_hbm.at[sidx.at[0]], xrows.at[0], semg)
        gr0 = pltpu.async_copy(r_hbm.at[eidx.at[0]], rrows.at[0], semg)
        ga1 = pltpu.async_copy(x_hbm.at[sidx.at[1]], xrows.at[1], semg)
        gr1 = pltpu.async_copy(r_hbm.at[eidx.at[1]], rrows.at[1], semg)
        ga0.wait()
        gr0.wait()
        compute(0)
        ga1.wait()
        gr1.wait()
        compute(1)
        return carry

    lax.fori_loop(0, NCHUNK // 2, pair, 0)
    plsc.subcore_barrier()
    ob = pl.multiple_of(c * NUM_ENT + s * ROWS_PT, 8)
    pltpu.sync_copy(acc.at[pl.ds(rb, ROWS_PT)], out_hbm.at[pl.ds(ob, ROWS_PT)])

    @pl.when(s == 0)
    def _():
        tb = NS * ROWS_PT
        otb = pl.multiple_of(c * NUM_ENT + tb, 8)
        pltpu.sync_copy(acc.at[pl.ds(tb, ROWS_TAIL)],
                        out_hbm.at[pl.ds(otb, ROWS_TAIL)])


@functools.lru_cache(maxsize=None)
def _edge_pass_fn():
    return pl.kernel(
        _edge_body,
        out_type=jax.ShapeDtypeStruct((2 * NUM_ENT, DIM), jnp.float32),
        mesh=plsc.VectorSubcoreMesh(core_axis_name="c", subcore_axis_name="s",
                                    num_cores=NC, num_subcores=NS),
        scratch_types=[
            pltpu.VMEM_SHARED((NUM_ENT, DIM), jnp.float32),
            pltpu.VMEM((2, CH, DIM), jnp.float32),
            pltpu.VMEM((2, CH, DIM), jnp.float32),
            pltpu.VMEM((CH, DIM), jnp.float32),
            pltpu.VMEM((2, CH), jnp.int32),
            pltpu.VMEM((2, CH), jnp.int32),
            pltpu.VMEM((2, CH), jnp.int32),
            pltpu.VMEM((2, CH), jnp.float32),
            pltpu.SemaphoreType.DMA,
            pltpu.SemaphoreType.DMA,
        ],
    )


def _edge_pass(*args):
    return _edge_pass_fn()(*args)


GPT = B // (NC * NS)


def _gather_body(x_hbm, r_hbm, subj_hbm, rel_hbm, sub_out, rel_out, idxv, rows, sem):
    c = lax.axis_index("c")
    s = lax.axis_index("s")
    base = pl.multiple_of((s * NC + c) * GPT, 8)
    pltpu.sync_copy(subj_hbm.at[pl.ds(base, GPT)], idxv)
    pltpu.async_copy(x_hbm.at[idxv], rows, sem).wait()
    pltpu.sync_copy(rows, sub_out.at[pl.ds(base, GPT)])
    pltpu.sync_copy(rel_hbm.at[pl.ds(base, GPT)], idxv)
    pltpu.async_copy(r_hbm.at[idxv], rows, sem).wait()
    pltpu.sync_copy(rows, rel_out.at[pl.ds(base, GPT)])


@functools.lru_cache(maxsize=None)
def _gather_pass_fn():
    return pl.kernel(
        _gather_body,
        out_type=[jax.ShapeDtypeStruct((B, DIM), jnp.float32),
                  jax.ShapeDtypeStruct((B, DIM), jnp.float32)],
        mesh=plsc.VectorSubcoreMesh(core_axis_name="c", subcore_axis_name="s",
                                    num_cores=NC, num_subcores=NS),
        scratch_types=[
            pltpu.VMEM((GPT,), jnp.int32),
            pltpu.VMEM((GPT, DIM), jnp.float32),
            pltpu.SemaphoreType.DMA,
        ],
    )


def _gather_pass(*args):
    return _gather_pass_fn()(*args)


def _dense_body(ain, aout, x, r, win, wout, wloop, wrel, lrel, b, bns, bno,
                xo, ro):
    f32 = jnp.float32
    agg = jnp.dot(ain[...], win[...], preferred_element_type=f32)
    agg = agg + jnp.dot(aout[...], wout[...], preferred_element_type=f32)
    loop = jnp.dot(x[...] * lrel[...], wloop[...], preferred_element_type=f32)
    h = agg * (1.0 / 3.0) + loop * (1.0 / 3.0) + b[...]
    mean = jnp.mean(h, axis=0, keepdims=True)
    var = jnp.mean((h - mean) * (h - mean), axis=0, keepdims=True)
    h = (h - mean) * lax.rsqrt(var + _EPS) * bns[...] + bno[...]
    xo[...] = jnp.tanh(h)
    ro[...] = jnp.dot(r[...], wrel[...], preferred_element_type=f32)


def _dense_layer(ain, aout, x, r, win, wout, wloop, wrel, lrel, b, bns, bno):
    return pl.pallas_call(
        _dense_body,
        out_shape=[jax.ShapeDtypeStruct((NUM_ENT, DIM), jnp.float32),
                   jax.ShapeDtypeStruct((2 * NUM_REL, DIM), jnp.float32)],
    )(ain, aout, x, r, win, wout, wloop, wrel,
      lrel.reshape(1, DIM), b.reshape(1, DIM),
      bns.reshape(1, DIM), bno.reshape(1, DIM))


def _d1_body(img, wbig, pool, stats, gstats):
    i = pl.program_id(0)
    blk = img[...]
    zraw = jnp.dot(blk, wbig[...], preferred_element_type=jnp.float32)
    s1 = jnp.sum(zraw, axis=0, keepdims=True)
    s2 = jnp.sum(zraw * zraw, axis=0, keepdims=True)
    s1f = jnp.dot(s1, pool[...], preferred_element_type=jnp.float32)
    s2f = jnp.dot(s2, pool[...], preferred_element_type=jnp.float32)
    part = jnp.concatenate([s1f, s2f], axis=0)
    gi1 = jnp.sum(blk)
    gi2 = jnp.sum(blk * blk)
    gpart = jnp.concatenate([jnp.full((1, DIM), gi1, jnp.float32),
                             jnp.full((1, DIM), gi2, jnp.float32)], axis=0)

    @pl.when(i == 0)
    def _():
        stats[...] = part
        gstats[...] = gpart

    @pl.when(i > 0)
    def _():
        stats[...] = stats[...] + part
        gstats[...] = gstats[...] + gpart


def _d1_call(img, wbig, pool):
    return pl.pallas_call(
        _d1_body,
        grid=(NBB,),
        in_specs=[pl.BlockSpec((BBLK, 2 * DIM), lambda i: (i, 0)),
                  pl.BlockSpec((2 * DIM, FLAT), lambda i: (0, 0)),
                  pl.BlockSpec((FLAT, NF), lambda i: (0, 0))],
        out_specs=[pl.BlockSpec((2, NF), lambda i: (0, 0)),
                   pl.BlockSpec((2, DIM), lambda i: (0, 0))],
        out_shape=[jax.ShapeDtypeStruct((2, NF), jnp.float32),
                   jax.ShapeDtypeStruct((2, DIM), jnp.float32)],
    )(img, wbig, pool)


def _d1b_body(pool, stats, gstats, ksum, cb, b0s, b0o, bcs, bco,
              scale_o, mix_o, a0_o):
    f32 = jnp.float32
    n0 = float(B * 2 * DIM)
    m0 = gstats[0, 0] / n0
    v0 = gstats[1, 0] / n0 - m0 * m0
    a0 = b0s[0, 0] * lax.rsqrt(v0 + _EPS)
    c0 = b0o[0, 0] - m0 * a0
    beta = c0 * ksum[...] + cb[...]
    nz = float(B * NPOS)
    mraw = stats[0:1, :] / nz
    vraw = stats[1:2, :] / nz - mraw * mraw
    mc = a0 * mraw + beta
    vc = (a0 * a0) * vraw
    scale_c = bcs[...] * lax.rsqrt(vc + _EPS)
    off_c = bco[...] - mc * scale_c
    mix = off_c + beta * scale_c
    scale_o[...] = lax.dot_general(scale_c, pool[...], (((1,), (1,)), ((), ())),
                                   preferred_element_type=f32)
    mix_o[...] = lax.dot_general(mix, pool[...], (((1,), (1,)), ((), ())),
                                 preferred_element_type=f32)
    a0_o[...] = jnp.full((1, DIM), a0, f32)


def _d1b_call(pool, stats, gstats, ksum, cb, b0s, b0o, bcs, bco):
    return pl.pallas_call(
        _d1b_body,
        out_shape=[jax.ShapeDtypeStruct((1, FLAT), jnp.float32),
                   jax.ShapeDtypeStruct((1, FLAT), jnp.float32),
                   jax.ShapeDtypeStruct((1, DIM), jnp.float32)],
    )(pool, stats, gstats, ksum.reshape(1, NF), cb.reshape(1, NF),
      b0s.reshape(1, 1), b0o.reshape(1, 1), bcs.reshape(1, NF),
      bco.reshape(1, NF))


def _d2_body(img, wbig, scale_f, mix_f, a0v, fw, fb, fo):
    f32 = jnp.float32
    blk = img[...]
    zraw = jnp.dot(blk, wbig[...], preferred_element_type=f32)
    z = (a0v[0, 0] * zraw) * scale_f[...] + mix_f[...]
    a = jnp.maximum(z, 0.0)
    fo[...] = jnp.dot(a, fw[...], preferred_element_type=f32) + fb[...]


def _d2_call(img, wbig, scale_f, mix_f, a0v, fw2, fb):
    return pl.pallas_call(
        _d2_body,
        grid=(NBB,),
        in_specs=[pl.BlockSpec((BBLK, 2 * DIM), lambda i: (i, 0)),
                  pl.BlockSpec((2 * DIM, FLAT), lambda i: (0, 0)),
                  pl.BlockSpec((1, FLAT), lambda i: (0, 0)),
                  pl.BlockSpec((1, FLAT), lambda i: (0, 0)),
                  pl.BlockSpec((1, DIM), lambda i: (0, 0)),
                  pl.BlockSpec((FLAT, DIM), lambda i: (0, 0)),
                  pl.BlockSpec((1, DIM), lambda i: (0, 0))],
        out_specs=pl.BlockSpec((BBLK, DIM), lambda i: (i, 0)),
        out_shape=jax.ShapeDtypeStruct((B, DIM), jnp.float32),
    )(img, wbig, scale_f, mix_f, a0v, fw2, fb.reshape(1, DIM))


def _d3_body(f, bfs, bfo, x2, eb, out):
    h = f[...]
    mean = jnp.mean(h, axis=0, keepdims=True)
    var = jnp.mean((h - mean) * (h - mean), axis=0, keepdims=True)
    h = (h - mean) * lax.rsqrt(var + _EPS) * bfs[...] + bfo[...]
    h = jnp.maximum(h, 0.0)
    sc = lax.dot_general(h, x2[...], (((1,), (1,)), ((), ())),
                         preferred_element_type=jnp.float32)
    out[...] = jax.nn.sigmoid(sc + eb[...])


def _d3_call(f, bfs, bfo, x2, ent_bias):
    return pl.pallas_call(
        _d3_body,
        grid=(NNB,),
        in_specs=[pl.BlockSpec((B, DIM), lambda i: (0, 0)),
                  pl.BlockSpec((1, DIM), lambda i: (0, 0)),
                  pl.BlockSpec((1, DIM), lambda i: (0, 0)),
                  pl.BlockSpec((NBLK, DIM), lambda i: (i, 0)),
                  pl.BlockSpec((1, NBLK), lambda i: (0, i))],
        out_specs=pl.BlockSpec((B, NBLK), lambda i: (0, i)),
        out_shape=jax.ShapeDtypeStruct((B, NUM_ENT), jnp.float32),
    )(f, bfs.reshape(1, DIM), bfo.reshape(1, DIM), x2, ent_bias.reshape(1, NUM_ENT))


def kernel(edge_index, edge_type, edge_norm, subj, rel, init_embed, init_rel,
           w_in1, w_out1, w_loop1, w_rel1, loop_rel1, b1, bn1_s, bn1_o,
           w_in2, w_out2, w_loop2, w_rel2, loop_rel2, b2, bn2_s, bn2_o,
           conv_w, conv_b, fc_w, fc_b, bn0_s, bn0_o, bnc_s, bnc_o,
           bnf_s, bnf_o, ent_bias):
    i32 = jnp.int32

    def prep(a, dt):
        ap = jnp.pad(a.astype(dt).reshape(2, E2), ((0, 0), (0, E2_P - E2)))
        return ap.reshape(2 * E2_P)

    src = prep(edge_index[0], i32)
    dst = prep(edge_index[1], i32)
    et = prep(edge_type, i32)
    nrm = prep(edge_norm, jnp.float32)
    zeros = jnp.zeros((NUM_ENT, DIM), jnp.float32)

    a1 = _edge_pass(init_embed, init_rel, src, et, dst, nrm, zeros)
    x1, r1 = _dense_layer(a1[:NUM_ENT], a1[NUM_ENT:], init_embed, init_rel,
                          w_in1, w_out1, w_loop1, w_rel1, loop_rel1, b1,
                          bn1_s, bn1_o)
    a2 = _edge_pass(x1, r1, src, et, dst, nrm, zeros)
    x2, r2 = _dense_layer(a2[:NUM_ENT], a2[NUM_ENT:], x1, r1,
                          w_in2, w_out2, w_loop2, w_rel2, loop_rel2, b2,
                          bn2_s, bn2_o)

    sub_e, rel_e = _gather_pass(x2, r2, subj.astype(i32), rel.astype(i32))
    img = jnp.concatenate([sub_e, rel_e], axis=1)
    km = conv_w.reshape(NF, KK)
    wbig = jnp.einsum("cpk,fk->cpf", _PATCH, km).reshape(2 * DIM, FLAT)
    pool = jnp.tile(jnp.eye(NF, dtype=jnp.float32), (NPOS, 1))
    ksum = jnp.sum(km, axis=1)
    fw2 = fc_w.reshape(NF, NPOS, DIM).transpose(1, 0, 2).reshape(FLAT, DIM)

    stats, gstats = _d1_call(img, wbig, pool)
    scale_f, mix_f, a0v = _d1b_call(pool, stats, gstats, ksum, conv_b,
                                    bn0_s, bn0_o, bnc_s, bnc_o)
    f = _d2_call(img, wbig, scale_f, mix_f, a0v, fw2, fc_b)
    return _d3_call(f, bnf_s, bnf_o, x2, ent_bias)
